# baseline (speedup 1.0000x reference)
import functools

import jax
import jax.numpy as jnp
from jax import lax
from jax.experimental import pallas as pl
from jax.experimental.pallas import tpu as pltpu
from jax.experimental.pallas import tpu_sc as plsc

_CA = 160
_B = 1024
_N = 50
_CW = 40
_NG = _B // 16
_BB = 128
_NBLK = _B // _BB
_TCL = 3


def _tc_body(idx_ref, oh_ref, *outs):
    oh = oh_ref[...]
    on_diag = (lax.broadcasted_iota(jnp.int32, (_CA, _CA), 0)
               == lax.broadcasted_iota(jnp.int32, (_CA, _CA), 1))
    diag2 = jnp.sum(jnp.where(on_diag, oh, 0.0), axis=1, keepdims=True)
    diag3 = diag2[None, :, :]
    iota_c = lax.broadcasted_iota(jnp.int32, (_N, _CA, _BB), 1)
    for l, o in enumerate(outs):
        idxv = idx_ref[l]
        eq = iota_c == idxv[:, None, :]
        o[...] = jnp.where(eq, diag3, 0.0)


def _tc_call(idx_t3, one_hot):
    shp = jax.ShapeDtypeStruct((_N, _CA, _B), jnp.float32)
    out_spec = pl.BlockSpec((_N, _CA, _BB), lambda i: (0, 0, i))
    return pl.pallas_call(
        _tc_body,
        grid=(_NBLK,),
        in_specs=[
            pl.BlockSpec((_TCL, _N, _BB), lambda i: (0, 0, i)),
            pl.BlockSpec((_CA, _CA), lambda i: (0, 0)),
        ],
        out_specs=[out_spec] * _TCL,
        out_shape=[shp] * _TCL,
    )(idx_t3, one_hot)


def _sc_body(idx_hbm, diag_hbm, zeros_hbm, out, idx_v, diag_v, buf0, buf1,
             sem0, sem1):
    bufs = (buf0, buf1)
    sems = (sem0, sem1)
    w = lax.axis_index("s") * 2 + lax.axis_index("c")
    iota = lax.iota(jnp.int32, 16)
    zeros16 = jnp.zeros((16,), jnp.float32)
    z16 = jnp.zeros((16,), jnp.int32)

    ng = w // 4
    c0 = lax.rem(w, 4) * _CW
    n0 = jnp.where(ng < 2, ng * 7, 14 + (ng - 2) * 6)
    nr = jnp.where(ng < 2, 7, 6)
    start = jnp.minimum(n0, _N - 7)
    off = n0 - start

    pltpu.sync_copy(idx_hbm.at[pl.ds(start * _B, 7 * _B)], idx_v)
    pltpu.sync_copy(diag_hbm, diag_v)
    pltpu.sync_copy(zeros_hbm, buf0)
    pltpu.sync_copy(zeros_hbm, buf1)

    def dma(ci, buf, sem):
        dst = out.at[pl.ds(n0 + ci, 1), pl.ds(c0, _CW), :]
        return pltpu.make_async_copy(buf, dst, sem)

    def fill(ci, buf, restore_ci):
        def grp(g, _):
            bvec = iota + g * 16
            if restore_ci is not None:
                ov = idx_v[pl.ds((restore_ci + off) * _B + g * 16, 16)]
                ocv = ov - c0
                om = (ocv >= 0) & (ocv < _CW)
                plsc.store_scatter(buf, [z16, ocv, bvec], zeros16, mask=om)
            idxvec = idx_v[pl.ds((ci + off) * _B + g * 16, 16)]
            cv = idxvec - c0
            m = (cv >= 0) & (cv < _CW)
            vals = plsc.load_gather(diag_v, [idxvec], mask=m)
            plsc.store_scatter(buf, [z16, cv, bvec], vals, mask=m)
            return 0
        lax.fori_loop(0, _NG, grp, 0)

    for b in range(2):
        fill(b, bufs[b], None)
        dma(b, bufs[b], sems[b]).start()

    def pair(gg, _):
        for b in range(2):
            ci = gg * 2 + b
            dma(ci - 2, bufs[b], sems[b]).wait()
            fill(ci, bufs[b], ci - 2)
            dma(ci, bufs[b], sems[b]).start()
        return 0
    lax.fori_loop(1, nr // 2, pair, 0)

    @pl.when(nr == 7)
    def _odd_tail():
        dma(4, bufs[0], sems[0]).wait()
        fill(6, bufs[0], 4)
        dma(6, bufs[0], sems[0]).start()

    dma(0, bufs[0], sems[0]).wait()
    dma(0, bufs[1], sems[1]).wait()


def _sc_call(idx_l3, diag, zeros):
    mesh = plsc.VectorSubcoreMesh(core_axis_name="c", subcore_axis_name="s")
    return pl.kernel(
        _sc_body,
        mesh=mesh,
        compiler_params=pltpu.CompilerParams(needs_layout_passes=False),
        out_type=jax.ShapeDtypeStruct((_N, _CA, _B), jnp.float32),
        scratch_types=[
            pltpu.VMEM((7 * _B,), jnp.int32),
            pltpu.VMEM((_CA,), jnp.float32),
            pltpu.VMEM((1, _CW, _B), jnp.float32),
            pltpu.VMEM((1, _CW, _B), jnp.float32),
            pltpu.SemaphoreType.DMA,
            pltpu.SemaphoreType.DMA,
        ])(idx_l3, diag, zeros)


def kernel(nei_rel_list, one_hot):
    idx_t = jnp.swapaxes(nei_rel_list, 1, 2)
    diag = jnp.diagonal(one_hot)
    zeros = jnp.zeros((1, _CW, _B), jnp.float32)
    sc_out = _sc_call(idx_t[_TCL].reshape(-1), diag, zeros)
    tc_outs = _tc_call(idx_t[:_TCL], one_hot)
    outs = tuple(tc_outs) + (sc_out,)
    return tuple(jnp.transpose(t, (2, 0, 1)) for t in outs)


# speedup vs baseline: 1.8166x; 1.8166x over previous
import jax
import jax.numpy as jnp
from jax import lax
from jax.experimental import pallas as pl

_CA = 160
_B = 1024
_N = 50
_BB = 128
_NBLK = _B // _BB


def _tc_body(idx_ref, oh_ref, o0, o1, o2, o3):
    oh = oh_ref[...]
    on_diag = (lax.broadcasted_iota(jnp.int32, (_CA, _CA), 0)
               == lax.broadcasted_iota(jnp.int32, (_CA, _CA), 1))
    diag2 = jnp.sum(jnp.where(on_diag, oh, 0.0), axis=1, keepdims=True)
    diag3 = diag2[None, :, :]
    iota_c = lax.broadcasted_iota(jnp.int32, (_N, _CA, _BB), 1)
    for l, o in enumerate((o0, o1, o2, o3)):
        idxv = idx_ref[l]
        eq = iota_c == idxv[:, None, :]
        o[...] = jnp.where(eq, diag3, 0.0)


def kernel(nei_rel_list, one_hot):
    idx_t = jnp.swapaxes(nei_rel_list, 1, 2)
    shp = jax.ShapeDtypeStruct((_N, _CA, _B), jnp.float32)
    out_spec = pl.BlockSpec((_N, _CA, _BB), lambda i: (0, 0, i))
    outs = pl.pallas_call(
        _tc_body,
        grid=(_NBLK,),
        in_specs=[
            pl.BlockSpec((4, _N, _BB), lambda i: (0, 0, i)),
            pl.BlockSpec((_CA, _CA), lambda i: (0, 0)),
        ],
        out_specs=[out_spec] * 4,
        out_shape=[shp] * 4,
    )(idx_t, one_hot)
    return tuple(jnp.transpose(t, (2, 0, 1)) for t in outs)
